# Initial kernel scaffold; baseline (speedup 1.0000x reference)
import jax
import jax.numpy as jnp
from jax.experimental import pallas as pl


def kernel(coords, vals, mask, W1, b1, W2, b2, W3, b3):
    raise NotImplementedError("write your pallas kernel here")


# speedup vs baseline: 5.1418x; 5.1418x over previous
import functools

import jax
import jax.numpy as jnp
from jax.experimental import pallas as pl

N_ORBS_C = 50
K_AGG_C = 4
BATCH = 8
NPTS = 1024
CH = 128


def _gigp_body(c11_ref, vals_ref, maskf_ref, W1_ref, b1_ref, W2_ref, b2_ref,
               W3_ref, b3_ref, out_ref):
    f32 = jnp.float32
    c11 = c11_ref[...]
    mn = jnp.min(c11)
    mx = jnp.max(c11)

    jj = jax.lax.broadcasted_iota(jnp.int32, (1, N_ORBS_C), 1).astype(f32)
    step = jj / f32(N_ORBS_C - 1)
    orbs = mn * (1.0 - step) + mx * step

    jidx = jax.lax.broadcasted_iota(jnp.int32, (NPTS, N_ORBS_C), 1)

    reprs = []
    for b in range(BATCH):
        cb = c11[b, :].reshape(NPTS, 1)
        d = jnp.abs(cb - orbs)
        sel = jnp.zeros((NPTS, N_ORBS_C), dtype=jnp.bool_)
        dw = d
        for _ in range(K_AGG_C):
            m = jnp.min(dw, axis=1, keepdims=True)
            ismin = dw == m
            j0 = jnp.min(jnp.where(ismin, jidx, N_ORBS_C), axis=1,
                         keepdims=True)
            onehot = jidx == j0
            sel = jnp.logical_or(sel, onehot)
            dw = jnp.where(onehot, jnp.inf, dw)
        M = sel.astype(f32)
        mv = jnp.where(maskf_ref[b, :].reshape(NPTS, 1) != 0.0,
                       vals_ref[b, :, :], 0.0)
        r = jax.lax.dot_general(
            M, mv, (((0,), (0,)), ((), ())),
            preferred_element_type=f32,
            precision=jax.lax.Precision.HIGHEST)
        reprs.append(r)

    orbs_repr = jnp.concatenate(reprs, axis=0)
    empty = jnp.sum(orbs_repr, axis=1, keepdims=True) == 0.0

    hp = jax.lax.Precision.HIGHEST
    h = jnp.maximum(
        jax.lax.dot_general(orbs_repr, W1_ref[...], (((1,), (0,)), ((), ())),
                            preferred_element_type=f32, precision=hp)
        + b1_ref[...].reshape(1, -1), 0.0)
    h = jnp.maximum(
        jax.lax.dot_general(h, W2_ref[...], (((1,), (0,)), ((), ())),
                            preferred_element_type=f32, precision=hp)
        + b2_ref[...].reshape(1, -1), 0.0)
    t = (jax.lax.dot_general(h, W3_ref[...], (((1,), (0,)), ((), ())),
                             preferred_element_type=f32, precision=hp)
         + b3_ref[...].reshape(1, -1))
    t = jnp.where(empty, 0.0, t)
    out_ref[...] = jnp.sum(t.reshape(BATCH, N_ORBS_C, CH), axis=1)


@jax.jit
def kernel(coords, vals, mask, W1, b1, W2, b2, W3, b3):
    c11 = coords[:, :, 1, 1]
    maskf = mask.astype(jnp.float32)
    out = pl.pallas_call(
        _gigp_body,
        out_shape=jax.ShapeDtypeStruct((BATCH, CH), jnp.float32),
    )(c11, vals, maskf, W1, b1, W2, b2, W3, b3)
    return out


# speedup vs baseline: 11.5096x; 2.2385x over previous
import functools

import jax
import jax.numpy as jnp
from jax.experimental import pallas as pl

N_ORBS_C = 50
K_AGG_C = 4
BATCH = 8
NPTS = 1024
CH = 128


def _gigp_body(c11_ref, vals_ref, maskf_ref, W1_ref, b1_ref, W2_ref, b2_ref,
               W3_ref, b3_ref, out_ref):
    f32 = jnp.float32
    c11 = c11_ref[...]
    mn = jnp.min(c11)
    mx = jnp.max(c11)
    delta = (mx - mn) / f32(N_ORBS_C - 1)

    jidx = jax.lax.broadcasted_iota(jnp.int32, (NPTS, N_ORBS_C), 1)

    reprs = []
    for b in range(BATCH):
        cb = c11[b, :].reshape(NPTS, 1)
        t = (cb - mn) / delta
        i0 = jnp.clip(jnp.floor(t).astype(jnp.int32) - 1, 0,
                      N_ORBS_C - K_AGG_C)
        M = jnp.logical_and(jidx >= i0, jidx < i0 + K_AGG_C).astype(f32)
        mv = jnp.where(maskf_ref[b, :].reshape(NPTS, 1) != 0.0,
                       vals_ref[b, :, :], 0.0)
        r = jax.lax.dot_general(
            M, mv, (((0,), (0,)), ((), ())),
            preferred_element_type=f32,
            precision=jax.lax.Precision.HIGHEST)
        reprs.append(r)

    orbs_repr = jnp.concatenate(reprs, axis=0)
    empty = jnp.sum(orbs_repr, axis=1, keepdims=True) == 0.0

    hp = jax.lax.Precision.HIGHEST
    h = jnp.maximum(
        jax.lax.dot_general(orbs_repr, W1_ref[...], (((1,), (0,)), ((), ())),
                            preferred_element_type=f32, precision=hp)
        + b1_ref[...].reshape(1, -1), 0.0)
    h = jnp.maximum(
        jax.lax.dot_general(h, W2_ref[...], (((1,), (0,)), ((), ())),
                            preferred_element_type=f32, precision=hp)
        + b2_ref[...].reshape(1, -1), 0.0)
    t = (jax.lax.dot_general(h, W3_ref[...], (((1,), (0,)), ((), ())),
                             preferred_element_type=f32, precision=hp)
         + b3_ref[...].reshape(1, -1))
    t = jnp.where(empty, 0.0, t)
    out_ref[...] = jnp.sum(t.reshape(BATCH, N_ORBS_C, CH), axis=1)


@jax.jit
def kernel(coords, vals, mask, W1, b1, W2, b2, W3, b3):
    c11 = coords[:, :, 1, 1]
    maskf = mask.astype(jnp.float32)
    out = pl.pallas_call(
        _gigp_body,
        out_shape=jax.ShapeDtypeStruct((BATCH, CH), jnp.float32),
    )(c11, vals, maskf, W1, b1, W2, b2, W3, b3)
    return out


# speedup vs baseline: 15.1717x; 1.3182x over previous
import functools

import jax
import jax.numpy as jnp
from jax.experimental import pallas as pl

N_ORBS_C = 50
K_AGG_C = 4
BATCH = 8
NPTS = 1024
CH = 128


def _gigp_body(c11_ref, vals_ref, maskf_ref, W1_ref, b1_ref, W2_ref, b2_ref,
               W3_ref, b3_ref, out_ref):
    f32 = jnp.float32
    c11 = c11_ref[...]
    mn = jnp.min(c11)
    mx = jnp.max(c11)
    delta = (mx - mn) / f32(N_ORBS_C - 1)

    jidx = jax.lax.broadcasted_iota(jnp.int32, (NPTS, N_ORBS_C), 1)

    reprs = []
    for b in range(BATCH):
        cb = c11[b, :].reshape(NPTS, 1)
        t = (cb - mn) / delta
        i0 = jnp.clip(jnp.floor(t).astype(jnp.int32) - 1, 0,
                      N_ORBS_C - K_AGG_C)
        M = jnp.logical_and(jidx >= i0, jidx < i0 + K_AGG_C).astype(f32)
        mv = jnp.where(maskf_ref[b, :].reshape(NPTS, 1) != 0.0,
                       vals_ref[b, :, :], 0.0)
        r = jax.lax.dot_general(
            M, mv, (((0,), (0,)), ((), ())),
            preferred_element_type=f32,
            precision=jax.lax.Precision.DEFAULT)
        reprs.append(r)

    orbs_repr = jnp.concatenate(reprs, axis=0)
    empty = jnp.sum(orbs_repr, axis=1, keepdims=True) == 0.0

    hp = jax.lax.Precision.DEFAULT
    h = jnp.maximum(
        jax.lax.dot_general(orbs_repr, W1_ref[...], (((1,), (0,)), ((), ())),
                            preferred_element_type=f32, precision=hp)
        + b1_ref[...].reshape(1, -1), 0.0)
    h = jnp.maximum(
        jax.lax.dot_general(h, W2_ref[...], (((1,), (0,)), ((), ())),
                            preferred_element_type=f32, precision=hp)
        + b2_ref[...].reshape(1, -1), 0.0)
    t = (jax.lax.dot_general(h, W3_ref[...], (((1,), (0,)), ((), ())),
                             preferred_element_type=f32, precision=hp)
         + b3_ref[...].reshape(1, -1))
    t = jnp.where(empty, 0.0, t)
    out_ref[...] = jnp.sum(t.reshape(BATCH, N_ORBS_C, CH), axis=1)


@jax.jit
def kernel(coords, vals, mask, W1, b1, W2, b2, W3, b3):
    c11 = coords[:, :, 1, 1]
    maskf = mask.astype(jnp.float32)
    out = pl.pallas_call(
        _gigp_body,
        out_shape=jax.ShapeDtypeStruct((BATCH, CH), jnp.float32),
    )(c11, vals, maskf, W1, b1, W2, b2, W3, b3)
    return out
